# baseline (speedup 1.0000x reference)
import functools

import jax
import jax.numpy as jnp
from jax import lax
from jax.experimental import pallas as pl
from jax.experimental.pallas import tpu as pltpu
from jax.experimental.pallas import tpu_sc as plsc

NG = 128
SUB = 16
CORES = 2
CH = 64

F32 = jnp.float32
I32 = jnp.int32


def _zero_rows(rows_ref, nrows, width):
    z16 = jnp.zeros((16,), F32)

    def body(r, _):
        for k in range(width // 16):
            rows_ref[r, pl.ds(k * 16, 16)] = z16
        return _

    lax.fori_loop(0, nrows, body, None)


NBUF = 4


def _pipelined_stage(h_ref, acc, idx_s, idx_d, rows, gsem, ssem, nch,
                     dacc=None, onesv=None, dsem=None):
    g = [None] * NBUF
    sc = [None] * NBUF
    dg = [None] * NBUF
    depth = min(NBUF - 1, nch)
    for j in range(depth):
        g[j] = pltpu.async_copy(h_ref.at[idx_s.at[j]], rows[j], gsem[j])
    for j in range(nch):
        b = j % NBUF
        g[b].wait()
        sc[b] = pltpu.async_copy(rows[b], acc.at[idx_d.at[j]], ssem[b],
                                 add=True)
        if dsem is not None:
            dg[b] = pltpu.async_copy(onesv, dacc.at[idx_d.at[j]], dsem[b],
                                     add=True)
        jn = j + depth
        if jn < nch:
            nb = jn % NBUF
            if jn - NBUF >= 0 or True:
                if sc[nb] is not None:
                    sc[nb].wait()
                    sc[nb] = None
                if dsem is not None and dg[nb] is not None:
                    dg[nb].wait()
                    dg[nb] = None
            g[nb] = pltpu.async_copy(h_ref.at[idx_s.at[jn]], rows[nb],
                                     gsem[nb])
    for b in range(NBUF):
        if sc[b] is not None:
            sc[b].wait()
        if dsem is not None and dg[b] is not None:
            dg[b].wait()


def _sc_segsum_layer0(src2d, dst2d, h0, NP, EP, D):
    rows_per_sub = NP // SUB
    chunks = EP // (CORES * SUB * CH)
    SCH = 32
    assert chunks % SCH == 0
    mesh = plsc.VectorSubcoreMesh(core_axis_name="c", subcore_axis_name="s")

    @functools.partial(
        pl.kernel,
        out_type=[
            jax.ShapeDtypeStruct((CORES * NP, D), F32),
            jax.ShapeDtypeStruct((CORES * NP,), F32),
        ],
        mesh=mesh,
        scratch_types=[
            pltpu.VMEM_SHARED((NP, D), F32),
            pltpu.VMEM_SHARED((NP,), F32),
            pltpu.VMEM((SCH, CH), I32),
            pltpu.VMEM((SCH, CH), I32),
            pltpu.VMEM((CH, D), F32),
            pltpu.VMEM((CH, D), F32),
            pltpu.VMEM((CH, D), F32),
            pltpu.VMEM((CH, D), F32),
            pltpu.VMEM((rows_per_sub,), F32),
            pltpu.VMEM((CH,), F32),
        ] + [pltpu.SemaphoreType.DMA] * 12,
    )
    def k(src_ref, dst_ref, h_ref, agg_ref, deg_ref,
          acc, dacc, idx_s, idx_d, rows0, rows1, rows2, rows3, zb1, onesv,
          gsem0, gsem1, gsem2, gsem3, ssem0, ssem1, ssem2, ssem3,
          dsem0, dsem1, dsem2, dsem3):
        rows = (rows0, rows1, rows2, rows3)
        gsem = (gsem0, gsem1, gsem2, gsem3)
        ssem = (ssem0, ssem1, ssem2, ssem3)
        dsem = (dsem0, dsem1, dsem2, dsem3)
        c = lax.axis_index("c")
        s = lax.axis_index("s")
        w = c * SUB + s

        _zero_rows(rows0, CH, D)

        def zb_body(j, _):
            zb1[pl.ds(pl.multiple_of(j * 16, 16), 16)] = jnp.zeros((16,), F32)
            return _
        lax.fori_loop(0, rows_per_sub // 16, zb_body, None)

        for k in range(CH // 16):
            onesv[pl.ds(k * 16, 16)] = jnp.ones((16,), F32)

        row0 = pl.multiple_of(s * rows_per_sub, CH)
        for k in range(rows_per_sub // CH):
            pltpu.sync_copy(rows0, acc.at[pl.ds(row0 + k * CH, CH)])
        pltpu.sync_copy(zb1, dacc.at[pl.ds(row0, rows_per_sub)])
        plsc.subcore_barrier()

        def stage(t, _):
            t0 = pl.multiple_of(t * SCH, SCH)
            pltpu.sync_copy(src_ref.at[w, pl.ds(t0, SCH)], idx_s)
            pltpu.sync_copy(dst_ref.at[w, pl.ds(t0, SCH)], idx_d)
            _pipelined_stage(h_ref, acc, idx_s, idx_d, rows, gsem, ssem,
                             SCH, dacc=dacc, onesv=onesv, dsem=dsem)
            return _
        lax.fori_loop(0, chunks // SCH, stage, None)

        plsc.subcore_barrier()
        out0 = pl.multiple_of(c * NP + s * rows_per_sub, CH)
        pltpu.sync_copy(acc.at[pl.ds(row0, rows_per_sub)],
                        agg_ref.at[pl.ds(out0, rows_per_sub)])
        pltpu.sync_copy(dacc.at[pl.ds(row0, rows_per_sub)],
                        deg_ref.at[pl.ds(out0, rows_per_sub)])

    return k(src2d, dst2d, h0)


def _sc_segsum_halves(src2_2d, dst2d, hs, NP, EP, D):
    rows_per_sub = NP // SUB
    chunks = EP // (SUB * CH)
    SCH = 32
    assert chunks % SCH == 0
    mesh = plsc.VectorSubcoreMesh(core_axis_name="c", subcore_axis_name="s")

    @functools.partial(
        pl.kernel,
        out_type=jax.ShapeDtypeStruct((CORES * NP, D), F32),
        mesh=mesh,
        scratch_types=[
            pltpu.VMEM_SHARED((NP, D), F32),
            pltpu.VMEM((SCH, CH), I32),
            pltpu.VMEM((SCH, CH), I32),
            pltpu.VMEM((CH, D), F32),
            pltpu.VMEM((CH, D), F32),
            pltpu.VMEM((CH, D), F32),
            pltpu.VMEM((CH, D), F32),
        ] + [pltpu.SemaphoreType.DMA] * 8,
    )
    def k(src_ref, dst_ref, h_ref, agg_ref, acc, idx_s, idx_d,
          rows0, rows1, rows2, rows3,
          gsem0, gsem1, gsem2, gsem3, ssem0, ssem1, ssem2, ssem3):
        rows = (rows0, rows1, rows2, rows3)
        gsem = (gsem0, gsem1, gsem2, gsem3)
        ssem = (ssem0, ssem1, ssem2, ssem3)
        c = lax.axis_index("c")
        s = lax.axis_index("s")

        _zero_rows(rows0, CH, D)
        row0 = pl.multiple_of(s * rows_per_sub, CH)
        for k in range(rows_per_sub // CH):
            pltpu.sync_copy(rows0, acc.at[pl.ds(row0 + k * CH, CH)])
        plsc.subcore_barrier()

        def stage(t, _):
            t0 = pl.multiple_of(t * SCH, SCH)
            pltpu.sync_copy(src_ref.at[c * SUB + s, pl.ds(t0, SCH)], idx_s)
            pltpu.sync_copy(dst_ref.at[s, pl.ds(t0, SCH)], idx_d)
            _pipelined_stage(h_ref, acc, idx_s, idx_d, rows, gsem, ssem, SCH)
            return _
        lax.fori_loop(0, chunks // SCH, stage, None)

        plsc.subcore_barrier()
        out0 = pl.multiple_of(c * NP + s * rows_per_sub, CH)
        pltpu.sync_copy(acc.at[pl.ds(row0, rows_per_sub)],
                        agg_ref.at[pl.ds(out0, rows_per_sub)])

    return k(src2_2d, dst2d, hs)


def _dot(a, b):
    return jax.lax.dot_general(a, b, (((1,), (0,)), ((), ())),
                               preferred_element_type=F32)


def _tc_embed(x3, emb, NP, BR, EMB, V):
    grid = NP // BR

    def body(x_ref, emb_ref, o_ref):
        xb = x_ref[0, 0, :]
        iota = lax.broadcasted_iota(I32, (BR, V), 1)
        oh = (xb[:, None] == iota).astype(F32)
        o_ref[...] = _dot(oh, emb_ref[...])

    return pl.pallas_call(
        body,
        grid=(grid,),
        in_specs=[
            pl.BlockSpec((1, 1, BR), lambda i: (i, 0, 0)),
            pl.BlockSpec((V, EMB), lambda i: (0, 0)),
        ],
        out_specs=pl.BlockSpec((BR, EMB), lambda i: (i, 0)),
        out_shape=jax.ShapeDtypeStruct((NP, EMB), F32),
    )(x3, emb)


def _tc_layer(agg, deg3, h, Wl, bl, Wr, NP, BR, first):
    grid = NP // BR
    Din = Wl.shape[0]
    HID = Wl.shape[1]
    HH = HID // 2

    def body(a_ref, d_ref, h_ref, wl_ref, bl_ref, wr_ref, o_ref):
        if first:
            agg_b = a_ref[0] + a_ref[1]
            h_b = h_ref[...]
        else:
            agg_b = jnp.concatenate([a_ref[0], a_ref[1]], axis=1)
            h_b = jnp.concatenate([h_ref[0], h_ref[1]], axis=1)
        deg = d_ref[0, 0, 0, :] + d_ref[1, 0, 0, :]
        di = 1.0 / jnp.maximum(deg, 1.0)
        z = _dot(agg_b * di[:, None], wl_ref[...]) + bl_ref[...]
        z = z + _dot(h_b, wr_ref[...])
        z = jnp.maximum(z, 0.0)
        o_ref[0] = z[:, :HH]
        o_ref[1] = z[:, HH:]

    h_spec = (pl.BlockSpec((BR, Din), lambda i: (i, 0)) if first
              else pl.BlockSpec((2, BR, HH), lambda i: (0, i, 0)))
    return pl.pallas_call(
        body,
        grid=(grid,),
        in_specs=[
            pl.BlockSpec((2, BR, Din if first else HH), lambda i: (0, i, 0)),
            pl.BlockSpec((2, 1, 1, BR), lambda i: (0, i, 0, 0)),
            h_spec,
            pl.BlockSpec((Din, HID), lambda i: (0, 0)),
            pl.BlockSpec((1, HID), lambda i: (0, 0)),
            pl.BlockSpec((Din, HID), lambda i: (0, 0)),
        ],
        out_specs=pl.BlockSpec((2, BR, HH), lambda i: (0, i, 0)),
        out_shape=jax.ShapeDtypeStruct((2, NP, HH), F32),
    )(agg, deg3, h, Wl, bl, Wr)


def _tc_pool(h3, batch3, Wout, bout, NP, BR):
    grid = NP // BR
    HID = Wout.shape[0]
    NC = Wout.shape[1]
    HH = HID // 2

    def body(h_ref, b_ref, wo_ref, bo_ref, o_ref, acc, cnt):
        i = pl.program_id(0)

        @pl.when(i == 0)
        def _():
            acc[...] = jnp.zeros_like(acc)
            cnt[...] = jnp.zeros_like(cnt)

        bb = b_ref[0, 0, :]
        iota = lax.broadcasted_iota(I32, (BR, NG), 1)
        P = (bb[:, None] == iota).astype(F32)
        hcat = jnp.concatenate([h_ref[0], h_ref[1]], axis=1)
        acc[...] += jax.lax.dot_general(P, hcat, (((0,), (0,)), ((), ())),
                                        preferred_element_type=F32)
        cnt[...] += jax.lax.dot_general(P, jnp.ones((BR, 8), F32),
                                        (((0,), (0,)), ((), ())),
                                        preferred_element_type=F32)

        @pl.when(i == grid - 1)
        def _():
            cn = jnp.maximum(cnt[:, 0:1], 1.0)
            pooled = acc[...] / cn
            o_ref[...] = _dot(pooled, wo_ref[...]) + bo_ref[...]

    return pl.pallas_call(
        body,
        grid=(grid,),
        in_specs=[
            pl.BlockSpec((2, BR, HH), lambda i: (0, i, 0)),
            pl.BlockSpec((1, 1, BR), lambda i: (i, 0, 0)),
            pl.BlockSpec((HID, NC), lambda i: (0, 0)),
            pl.BlockSpec((1, NC), lambda i: (0, 0)),
        ],
        out_specs=pl.BlockSpec((NG, NC), lambda i: (0, 0)),
        out_shape=jax.ShapeDtypeStruct((NG, NC), F32),
        scratch_shapes=[
            pltpu.VMEM((NG, HID), F32),
            pltpu.VMEM((NG, 8), F32),
        ],
    )(h3, batch3, Wout, bout)


def kernel(x, edge_index, batch, emb, Wl0, bl0, Wr0, Wl1, bl1, Wr1,
           Wl2, bl2, Wr2, Wout, bout):
    N = x.shape[0]
    E = edge_index.shape[1]
    V, EMB = emb.shape
    HID = Wl0.shape[1]
    HH = HID // 2

    NP = ((N + SUB * CH - 1) // (SUB * CH)) * (SUB * CH)
    EQ = CORES * SUB * CH * 32
    EP = ((E + EQ - 1) // EQ) * EQ
    BR = 320
    assert NP % BR == 0

    src = edge_index[0].astype(I32)
    dst = edge_index[1].astype(I32)
    padE = EP - E
    srcp = jnp.concatenate([src, jnp.full((padE,), N, I32)])
    dstp = jnp.concatenate([dst, jnp.full((padE,), NP - 1, I32)])
    nw = CORES * SUB
    src2d = srcp.reshape(nw, EP // (nw * CH), CH)
    dst2d = dstp.reshape(nw, EP // (nw * CH), CH)
    src2_2d = jnp.concatenate([srcp, srcp + NP]).reshape(
        nw, EP // (SUB * CH), CH)
    dstB = dstp.reshape(SUB, EP // (SUB * CH), CH)

    x3 = jnp.pad(x.astype(I32), (0, NP - N)).reshape(NP // BR, 1, BR)
    batch3 = jnp.concatenate(
        [batch.astype(I32), jnp.full((NP - N,), NG, I32)]
    ).reshape(NP // BR, 1, BR)

    bl0r, bl1r, bl2r = (b.reshape(1, HID) for b in (bl0, bl1, bl2))
    boutr = bout.reshape(1, -1)

    h0 = _tc_embed(x3, emb, NP, BR, EMB, V)

    agg0_flat, deg_flat = _sc_segsum_layer0(src2d, dst2d, h0, NP, EP, EMB)
    agg0 = agg0_flat.reshape(2, NP, EMB)
    deg3 = deg_flat.reshape(2, NP // BR, 1, BR)

    h1 = _tc_layer(agg0, deg3, h0, Wl0, bl0r, Wr0, NP, BR, first=True)
    agg1 = _sc_segsum_halves(src2_2d, dstB, h1.reshape(2 * NP, HH),
                             NP, EP, HH).reshape(2, NP, HH)
    h2 = _tc_layer(agg1, deg3, h1, Wl1, bl1r, Wr1, NP, BR, first=False)
    agg2 = _sc_segsum_halves(src2_2d, dstB, h2.reshape(2 * NP, HH),
                             NP, EP, HH).reshape(2, NP, HH)
    h3 = _tc_layer(agg2, deg3, h2, Wl2, bl2r, Wr2, NP, BR, first=False)

    return _tc_pool(h3, batch3, Wout, boutr, NP, BR)


# speedup vs baseline: 1.2535x; 1.2535x over previous
import functools

import jax
import jax.numpy as jnp
from jax import lax
from jax.experimental import pallas as pl
from jax.experimental.pallas import tpu as pltpu
from jax.experimental.pallas import tpu_sc as plsc

NG = 128
SUB = 16
CORES = 2
CH = 64
NBUF = 4
SCH = 32

F32 = jnp.float32
I32 = jnp.int32
BF16 = jnp.bfloat16

_HI_MASK = -65536


def _zero_rows(rows_ref, nrows, width):
    z16 = jnp.zeros((16,), F32)

    def body(r, _):
        for k in range(width // 16):
            rows_ref[r, pl.ds(k * 16, 16)] = z16
        return _

    lax.fori_loop(0, nrows, body, None)


def _widen_rows(src_ref, dst_ref, nrows, nwords):
    def body(r2, _):
        for dr in range(2):
            r = r2 * 2 + dr
            for k in range(nwords // 16):
                u = src_ref[r, pl.ds(k * 16, 16)]
                dst_ref[r, pl.ds(k * 16, 16)] = jax.lax.bitcast_convert_type(
                    u << 16, F32)
                dst_ref[r, pl.ds(nwords + k * 16, 16)] = (
                    jax.lax.bitcast_convert_type(u, F32))
        return _

    lax.fori_loop(0, nrows // 2, body, None)


def _ring_stage(h_ref, acc, idx_s, idx_d, rows, conv, gsem, ssem, nch, D,
                dacc=None, onesv=None, dsem=None):
    g = [None] * NBUF
    sc = [None, None]
    dg = [None, None]
    depth = min(NBUF, nch)
    for j in range(depth):
        g[j] = pltpu.async_copy(h_ref.at[idx_s.at[j]], rows[j], gsem[j])
    for j in range(nch):
        b = j % NBUF
        cb = j & 1
        g[b].wait()
        if sc[cb] is not None:
            sc[cb].wait()
            sc[cb] = None
        if dsem is not None and dg[cb] is not None:
            dg[cb].wait()
            dg[cb] = None
        _widen_rows(rows[b], conv[cb], CH, D // 2)
        if j + NBUF < nch:
            g[b] = pltpu.async_copy(h_ref.at[idx_s.at[j + NBUF]], rows[b],
                                    gsem[b])
        sc[cb] = pltpu.async_copy(conv[cb], acc.at[idx_d.at[j]], ssem[cb],
                                  add=True)
        if dsem is not None:
            dg[cb] = pltpu.async_copy(onesv, dacc.at[idx_d.at[j]], dsem[cb],
                                      add=True)
    for b in (0, 1):
        if sc[b] is not None:
            sc[b].wait()
        if dsem is not None and dg[b] is not None:
            dg[b].wait()


def _sc_segsum_layer0(src3, dst3, h0p, NP, EP, D):
    rows_per_sub = NP // SUB
    chunks = EP // (CORES * SUB * CH)
    assert chunks % SCH == 0
    mesh = plsc.VectorSubcoreMesh(core_axis_name="c", subcore_axis_name="s")

    @functools.partial(
        pl.kernel,
        out_type=[
            jax.ShapeDtypeStruct((CORES * NP, D), F32),
            jax.ShapeDtypeStruct((CORES * NP,), F32),
        ],
        mesh=mesh,
        scratch_types=[
            pltpu.VMEM_SHARED((NP, D), F32),
            pltpu.VMEM_SHARED((NP,), F32),
            pltpu.VMEM((SCH, CH), I32),
            pltpu.VMEM((SCH, CH), I32),
            pltpu.VMEM((CH, D // 2), I32),
            pltpu.VMEM((CH, D // 2), I32),
            pltpu.VMEM((CH, D // 2), I32),
            pltpu.VMEM((CH, D // 2), I32),
            pltpu.VMEM((CH, D), F32),
            pltpu.VMEM((CH, D), F32),
            pltpu.VMEM((rows_per_sub,), F32),
            pltpu.VMEM((CH,), F32),
        ] + [pltpu.SemaphoreType.DMA] * 8,
        compiler_params=pltpu.CompilerParams(use_tc_tiling_on_sc=False),
    )
    def k(src_ref, dst_ref, h_ref, agg_ref, deg_ref,
          acc, dacc, idx_s, idx_d, rows0, rows1, rows2, rows3,
          conv0, conv1, zb1, onesv,
          gsem0, gsem1, gsem2, gsem3, ssem0, ssem1, dsem0, dsem1):
        rows = (rows0, rows1, rows2, rows3)
        conv = (conv0, conv1)
        gsem = (gsem0, gsem1, gsem2, gsem3)
        ssem = (ssem0, ssem1)
        dsem = (dsem0, dsem1)
        c = lax.axis_index("c")
        s = lax.axis_index("s")
        w = c * SUB + s

        _zero_rows(conv0, CH, D)

        def zb_body(j, _):
            zb1[pl.ds(pl.multiple_of(j * 16, 16), 16)] = jnp.zeros((16,), F32)
            return _
        lax.fori_loop(0, rows_per_sub // 16, zb_body, None)

        for k in range(CH // 16):
            onesv[pl.ds(k * 16, 16)] = jnp.ones((16,), F32)

        row0 = pl.multiple_of(s * rows_per_sub, CH)
        for k in range(rows_per_sub // CH):
            pltpu.sync_copy(conv0, acc.at[pl.ds(row0 + k * CH, CH)])
        pltpu.sync_copy(zb1, dacc.at[pl.ds(row0, rows_per_sub)])
        plsc.subcore_barrier()

        def stage(t, _):
            t0 = pl.multiple_of(t * SCH, SCH)
            pltpu.sync_copy(src_ref.at[w, pl.ds(t0, SCH)], idx_s)
            pltpu.sync_copy(dst_ref.at[w, pl.ds(t0, SCH)], idx_d)
            _ring_stage(h_ref, acc, idx_s, idx_d, rows, conv, gsem, ssem,
                        SCH, D, dacc=dacc, onesv=onesv, dsem=dsem)
            return _
        lax.fori_loop(0, chunks // SCH, stage, None)

        plsc.subcore_barrier()
        out0 = pl.multiple_of(c * NP + s * rows_per_sub, CH)
        pltpu.sync_copy(acc.at[pl.ds(row0, rows_per_sub)],
                        agg_ref.at[pl.ds(out0, rows_per_sub)])
        pltpu.sync_copy(dacc.at[pl.ds(row0, rows_per_sub)],
                        deg_ref.at[pl.ds(out0, rows_per_sub)])

    return k(src3, dst3, h0p)


def _sc_segsum_halves(src2_3, dst3, hsp, NP, EP, D):
    rows_per_sub = NP // SUB
    chunks = EP // (SUB * CH)
    assert chunks % SCH == 0
    mesh = plsc.VectorSubcoreMesh(core_axis_name="c", subcore_axis_name="s")

    @functools.partial(
        pl.kernel,
        out_type=jax.ShapeDtypeStruct((CORES * NP, D), F32),
        mesh=mesh,
        scratch_types=[
            pltpu.VMEM_SHARED((NP, D), F32),
            pltpu.VMEM((SCH, CH), I32),
            pltpu.VMEM((SCH, CH), I32),
            pltpu.VMEM((CH, D // 2), I32),
            pltpu.VMEM((CH, D // 2), I32),
            pltpu.VMEM((CH, D // 2), I32),
            pltpu.VMEM((CH, D // 2), I32),
            pltpu.VMEM((CH, D), F32),
            pltpu.VMEM((CH, D), F32),
        ] + [pltpu.SemaphoreType.DMA] * 6,
        compiler_params=pltpu.CompilerParams(use_tc_tiling_on_sc=False),
    )
    def k(src_ref, dst_ref, h_ref, agg_ref, acc, idx_s, idx_d,
          rows0, rows1, rows2, rows3, conv0, conv1,
          gsem0, gsem1, gsem2, gsem3, ssem0, ssem1):
        rows = (rows0, rows1, rows2, rows3)
        conv = (conv0, conv1)
        gsem = (gsem0, gsem1, gsem2, gsem3)
        ssem = (ssem0, ssem1)
        c = lax.axis_index("c")
        s = lax.axis_index("s")

        _zero_rows(conv0, CH, D)
        row0 = pl.multiple_of(s * rows_per_sub, CH)
        for k in range(rows_per_sub // CH):
            pltpu.sync_copy(conv0, acc.at[pl.ds(row0 + k * CH, CH)])
        plsc.subcore_barrier()

        def stage(t, _):
            t0 = pl.multiple_of(t * SCH, SCH)
            pltpu.sync_copy(src_ref.at[c * SUB + s, pl.ds(t0, SCH)], idx_s)
            pltpu.sync_copy(dst_ref.at[s, pl.ds(t0, SCH)], idx_d)
            _ring_stage(h_ref, acc, idx_s, idx_d, rows, conv, gsem, ssem,
                        SCH, D)
            return _
        lax.fori_loop(0, chunks // SCH, stage, None)

        plsc.subcore_barrier()
        out0 = pl.multiple_of(c * NP + s * rows_per_sub, CH)
        pltpu.sync_copy(acc.at[pl.ds(row0, rows_per_sub)],
                        agg_ref.at[pl.ds(out0, rows_per_sub)])

    return k(src2_3, dst3, hsp)


def _dot(a, b):
    return jax.lax.dot_general(a, b, (((1,), (0,)), ((), ())),
                               preferred_element_type=F32)


def _pack_bf16(z):
    hw = z.shape[1] // 2
    rnd = z.astype(BF16).astype(F32)
    lo = jax.lax.bitcast_convert_type(rnd[:, :hw], I32)
    hi = jax.lax.bitcast_convert_type(rnd[:, hw:], I32)
    return (jax.lax.shift_right_logical(lo, 16) | (hi & _HI_MASK))


def _tc_embed(x3, emb, NP, BR, EMB, V):
    grid = NP // BR

    def body(x_ref, emb_ref, o_ref, op_ref):
        xb = x_ref[0, 0, :]
        iota = lax.broadcasted_iota(I32, (BR, V), 1)
        oh = (xb[:, None] == iota).astype(F32)
        h = _dot(oh, emb_ref[...])
        o_ref[...] = h
        op_ref[...] = _pack_bf16(h)

    return pl.pallas_call(
        body,
        grid=(grid,),
        in_specs=[
            pl.BlockSpec((1, 1, BR), lambda i: (i, 0, 0)),
            pl.BlockSpec((V, EMB), lambda i: (0, 0)),
        ],
        out_specs=[
            pl.BlockSpec((BR, EMB), lambda i: (i, 0)),
            pl.BlockSpec((BR, EMB // 2), lambda i: (i, 0)),
        ],
        out_shape=[
            jax.ShapeDtypeStruct((NP, EMB), F32),
            jax.ShapeDtypeStruct((NP, EMB // 2), I32),
        ],
    )(x3, emb)


def _tc_layer(agg, deg3, h, Wl, bl, Wr, NP, BR, first, emit_packed=True):
    grid = NP // BR
    Din = Wl.shape[0]
    HID = Wl.shape[1]
    HH = HID // 2

    def body(a_ref, d_ref, h_ref, wl_ref, bl_ref, wr_ref, o_ref, op_ref):
        if first:
            agg_b = a_ref[0] + a_ref[1]
            h_b = h_ref[...]
        else:
            agg_b = jnp.concatenate([a_ref[0], a_ref[1]], axis=1)
            h_b = jnp.concatenate([h_ref[0], h_ref[1]], axis=1)
        deg = d_ref[0, 0, 0, :] + d_ref[1, 0, 0, :]
        di = 1.0 / jnp.maximum(deg, 1.0)
        z = _dot(agg_b * di[:, None], wl_ref[...]) + bl_ref[...]
        z = z + _dot(h_b, wr_ref[...])
        z = jnp.maximum(z, 0.0)
        o_ref[0] = z[:, :HH]
        o_ref[1] = z[:, HH:]
        if emit_packed:
            op_ref[0] = _pack_bf16(z[:, :HH])
            op_ref[1] = _pack_bf16(z[:, HH:])

    h_spec = (pl.BlockSpec((BR, Din), lambda i: (i, 0)) if first
              else pl.BlockSpec((2, BR, HH), lambda i: (0, i, 0)))
    out_specs = [pl.BlockSpec((2, BR, HH), lambda i: (0, i, 0)),
                 pl.BlockSpec((2, BR, HH // 2), lambda i: (0, i, 0))]
    out_shape = [jax.ShapeDtypeStruct((2, NP, HH), F32),
                 jax.ShapeDtypeStruct((2, NP, HH // 2), I32)]
    return pl.pallas_call(
        body,
        grid=(grid,),
        in_specs=[
            pl.BlockSpec((2, BR, Din if first else HH), lambda i: (0, i, 0)),
            pl.BlockSpec((2, 1, 1, BR), lambda i: (0, i, 0, 0)),
            h_spec,
            pl.BlockSpec((Din, HID), lambda i: (0, 0)),
            pl.BlockSpec((1, HID), lambda i: (0, 0)),
            pl.BlockSpec((Din, HID), lambda i: (0, 0)),
        ],
        out_specs=out_specs,
        out_shape=out_shape,
    )(agg, deg3, h, Wl, bl, Wr)


def _tc_pool(h3, batch3, Wout, bout, NP, BR):
    grid = NP // BR
    HID = Wout.shape[0]
    NC = Wout.shape[1]
    HH = HID // 2

    def body(h_ref, b_ref, wo_ref, bo_ref, o_ref, acc, cnt):
        i = pl.program_id(0)

        @pl.when(i == 0)
        def _():
            acc[...] = jnp.zeros_like(acc)
            cnt[...] = jnp.zeros_like(cnt)

        bb = b_ref[0, 0, :]
        iota = lax.broadcasted_iota(I32, (BR, NG), 1)
        P = (bb[:, None] == iota).astype(F32)
        hcat = jnp.concatenate([h_ref[0], h_ref[1]], axis=1)
        acc[...] += jax.lax.dot_general(P, hcat, (((0,), (0,)), ((), ())),
                                        preferred_element_type=F32)
        cnt[...] += jax.lax.dot_general(P, jnp.ones((BR, 8), F32),
                                        (((0,), (0,)), ((), ())),
                                        preferred_element_type=F32)

        @pl.when(i == grid - 1)
        def _():
            cn = jnp.maximum(cnt[:, 0:1], 1.0)
            pooled = acc[...] / cn
            o_ref[...] = _dot(pooled, wo_ref[...]) + bo_ref[...]

    return pl.pallas_call(
        body,
        grid=(grid,),
        in_specs=[
            pl.BlockSpec((2, BR, HH), lambda i: (0, i, 0)),
            pl.BlockSpec((1, 1, BR), lambda i: (i, 0, 0)),
            pl.BlockSpec((HID, NC), lambda i: (0, 0)),
            pl.BlockSpec((1, NC), lambda i: (0, 0)),
        ],
        out_specs=pl.BlockSpec((NG, NC), lambda i: (0, 0)),
        out_shape=jax.ShapeDtypeStruct((NG, NC), F32),
        scratch_shapes=[
            pltpu.VMEM((NG, HID), F32),
            pltpu.VMEM((NG, 8), F32),
        ],
    )(h3, batch3, Wout, bout)


def kernel(x, edge_index, batch, emb, Wl0, bl0, Wr0, Wl1, bl1, Wr1,
           Wl2, bl2, Wr2, Wout, bout):
    N = x.shape[0]
    E = edge_index.shape[1]
    V, EMB = emb.shape
    HID = Wl0.shape[1]
    HH = HID // 2

    NP = ((N + SUB * 128 - 1) // (SUB * 128)) * (SUB * 128)
    EQ = CORES * SUB * CH * SCH
    EP = ((E + EQ - 1) // EQ) * EQ
    BR = 320
    assert NP % BR == 0

    src = edge_index[0].astype(I32)
    dst = edge_index[1].astype(I32)
    padE = EP - E
    srcp = jnp.concatenate([src, jnp.full((padE,), N, I32)])
    dstp = jnp.concatenate([dst, jnp.full((padE,), NP - 1, I32)])
    nw = CORES * SUB
    src3 = srcp.reshape(nw, EP // (nw * CH), CH)
    dst3 = dstp.reshape(nw, EP // (nw * CH), CH)
    src2_3 = jnp.concatenate([srcp, srcp + NP]).reshape(
        nw, EP // (SUB * CH), CH)
    dstB = dstp.reshape(SUB, EP // (SUB * CH), CH)

    x3 = jnp.pad(x.astype(I32), (0, NP - N)).reshape(NP // BR, 1, BR)
    batch3 = jnp.concatenate(
        [batch.astype(I32), jnp.full((NP - N,), NG, I32)]
    ).reshape(NP // BR, 1, BR)

    bl0r, bl1r, bl2r = (b.reshape(1, HID) for b in (bl0, bl1, bl2))
    boutr = bout.reshape(1, -1)

    h0, h0p = _tc_embed(x3, emb, NP, BR, EMB, V)

    agg0_flat, deg_flat = _sc_segsum_layer0(src3, dst3, h0p, NP, EP, EMB)
    agg0 = agg0_flat.reshape(2, NP, EMB)
    deg3 = deg_flat.reshape(2, NP // BR, 1, BR)

    h1, h1p = _tc_layer(agg0, deg3, h0, Wl0, bl0r, Wr0, NP, BR, first=True)
    agg1 = _sc_segsum_halves(src2_3, dstB, h1p.reshape(2 * NP, HH // 2),
                             NP, EP, HH).reshape(2, NP, HH)
    h2, h2p = _tc_layer(agg1, deg3, h1, Wl1, bl1r, Wr1, NP, BR, first=False)
    agg2 = _sc_segsum_halves(src2_3, dstB, h2p.reshape(2 * NP, HH // 2),
                             NP, EP, HH).reshape(2, NP, HH)
    h3, _ = _tc_layer(agg2, deg3, h2, Wl2, bl2r, Wr2, NP, BR, first=False)

    return _tc_pool(h3, batch3, Wout, boutr, NP, BR)
